# Initial kernel scaffold; baseline (speedup 1.0000x reference)
import jax
import jax.numpy as jnp
from jax.experimental import pallas as pl


def kernel(t, W):
    raise NotImplementedError("write your pallas kernel here")


# speedup vs baseline: 7.9742x; 7.9742x over previous
import functools

import jax
import jax.numpy as jnp
from jax import lax
from jax.experimental import pallas as pl
from jax.experimental.pallas import tpu as pltpu
from jax.experimental.pallas import tpu_sc as plsc

NUM_ROWS = 100000
DIM = 32
KH = 4
HASH_C = 73244475
C0 = HASH_C & 0xFFFF
C1 = HASH_C >> 16

NC = 2
NS = 16
NW = NC * NS

N_TOK = 16384 * 26
TOK_W = N_TOK // NW
BLK = 128
NBLK = TOK_W // BLK


def _i32(x):
    return jnp.int32(x)


def _lsr16(x):
    return lax.shift_right_logical(x, _i32(16))


def _mul_c(hi, lo):
    a0 = lax.bitwise_and(lo, _i32(0xFFFF))
    a1 = _lsr16(lo)
    p00 = a0 * _i32(C0)
    p01 = a0 * _i32(C1)
    p10 = a1 * _i32(C0)
    p11 = a1 * _i32(C1)
    mid = p10 + _lsr16(p00)
    mid2 = p01 + lax.bitwise_and(mid, _i32(0xFFFF))
    lo_out = lax.bitwise_or(lax.shift_left(mid2, _i32(16)),
                            lax.bitwise_and(p00, _i32(0xFFFF)))
    high = p11 + _lsr16(mid) + _lsr16(mid2)
    hi_out = hi * _i32(HASH_C) + high
    return hi_out, lo_out


def _shr16(hi, lo):
    new_lo = lax.bitwise_or(_lsr16(lo), lax.shift_left(hi, _i32(16)))
    return lax.shift_right_arithmetic(hi, _i32(16)), new_lo


def _hash_idx(lo0):
    a = lax.bitwise_xor(_lsr16(lo0), lo0)
    zero = jnp.zeros_like(lo0)
    hi, lo = _mul_c(zero, a)
    sh, sl = _shr16(hi, lo)
    hi, lo = _mul_c(lax.bitwise_xor(sh, hi), lax.bitwise_xor(sl, lo))
    sh, sl = _shr16(hi, lo)
    hi = lax.bitwise_xor(sh, hi)
    lo = lax.bitwise_xor(sl, lo)
    m = _i32(100000)
    hi_m = lax.rem(hi, m)
    hi_m = jnp.where(hi_m < 0, hi_m + m, hi_m)
    t1 = lax.rem(hi_m * _i32(296), m)
    t2 = lax.rem(hi_m * _i32(67), _i32(100)) * _i32(1000)
    lo_pos = lax.bitwise_and(lo, _i32(0x7FFFFFFF))
    lo_m = lax.rem(lo_pos, m) + jnp.where(lo < 0, _i32(83648), _i32(0))
    return lax.rem(t1 + t2 + lo_m, m)


@functools.partial(
    pl.kernel,
    mesh=plsc.VectorSubcoreMesh(core_axis_name="c", subcore_axis_name="s"),
    out_type=jax.ShapeDtypeStruct((N_TOK, DIM), jnp.float32),
    compiler_params=pltpu.CompilerParams(use_tc_tiling_on_sc=False),
    scratch_types=[
        pltpu.VMEM((TOK_W,), jnp.int32),
        pltpu.VMEM((KH, NBLK, BLK), jnp.int32),
        pltpu.VMEM((KH, BLK, DIM), jnp.float32),
        pltpu.VMEM((BLK, DIM), jnp.float32),
        pltpu.SemaphoreType.DMA,
    ],
)
def _sc_embed(t_hbm, w_hbm, out_hbm, t_v, idx_v, gbuf, acc, sem):
    wid = lax.axis_index("s") * NC + lax.axis_index("c")
    base = wid * TOK_W

    pltpu.sync_copy(t_hbm.at[pl.ds(base, TOK_W)], t_v)

    def hash_body(ci, _):
        tv = t_v[pl.ds(ci * 16, 16)]
        bb = lax.div(ci, _i32(8))
        kk = lax.rem(ci, _i32(8))
        for r in range(KH):
            idx_v[_i32(r), bb, pl.ds(kk * 16, 16)] = _hash_idx(tv + _i32(r))
        return 0

    lax.fori_loop(_i32(0), _i32(TOK_W // 16), hash_body, 0)

    def block_body(b, _):
        copies = [
            pltpu.async_copy(w_hbm.at[idx_v.at[_i32(r), b]], gbuf.at[_i32(r)],
                             sem)
            for r in range(KH)
        ]
        for cp in copies:
            cp.wait()

        def reduce_body(i, _):
            for h in (0, 16):
                s = pl.ds(h, 16)
                v = (gbuf[_i32(0), i, s] + gbuf[_i32(1), i, s]
                     + gbuf[_i32(2), i, s] + gbuf[_i32(3), i, s])
                acc[i, s] = v * 0.25
            return 0

        lax.fori_loop(_i32(0), _i32(BLK), reduce_body, 0)
        pltpu.sync_copy(acc, out_hbm.at[pl.ds(base + b * BLK, BLK)])
        return 0

    lax.fori_loop(_i32(0), _i32(NBLK), block_body, 0)


@jax.jit
def kernel(t, W):
    t32 = t.reshape(-1).astype(jnp.int32)
    out = _sc_embed(t32, W)
    return out.reshape(t.shape + (DIM,))


# speedup vs baseline: 13.4264x; 1.6837x over previous
import functools

import jax
import jax.numpy as jnp
from jax import lax
from jax.experimental import pallas as pl
from jax.experimental.pallas import tpu as pltpu
from jax.experimental.pallas import tpu_sc as plsc

NUM_ROWS = 100000
DIM = 32
KH = 4
HASH_C = 73244475
C0 = HASH_C & 0xFFFF
C1 = HASH_C >> 16

MAGIC = (1 << 47) // 100000 + 1
MG0 = MAGIC & 0xFFFF
MG1 = MAGIC >> 16

NC = 2
NS = 16
NW = NC * NS

N_TOK = 16384 * 26
TOK_W = N_TOK // NW
BLK = 128
NBLK = TOK_W // BLK


def _i32(x):
    return jnp.int32(x)


def _lsr16(x):
    return lax.shift_right_logical(x, _i32(16))


def _mul_c(hi, lo):
    a0 = lax.bitwise_and(lo, _i32(0xFFFF))
    a1 = _lsr16(lo)
    p00 = a0 * _i32(C0)
    p01 = a0 * _i32(C1)
    p10 = a1 * _i32(C0)
    p11 = a1 * _i32(C1)
    mid = p10 + _lsr16(p00)
    mid2 = p01 + lax.bitwise_and(mid, _i32(0xFFFF))
    lo_out = lax.bitwise_or(lax.shift_left(mid2, _i32(16)),
                            lax.bitwise_and(p00, _i32(0xFFFF)))
    high = p11 + _lsr16(mid) + _lsr16(mid2)
    hi_out = hi * _i32(HASH_C) + high
    return hi_out, lo_out


def _shr16(hi, lo):
    new_lo = lax.bitwise_or(_lsr16(lo), lax.shift_left(hi, _i32(16)))
    return lax.shift_right_arithmetic(hi, _i32(16)), new_lo


def _mulhi31(x, m0, m1):
    a0 = lax.bitwise_and(x, _i32(0xFFFF))
    a1 = _lsr16(x)
    p00 = a0 * _i32(m0)
    p01 = a0 * _i32(m1)
    p10 = a1 * _i32(m0)
    p11 = a1 * _i32(m1)
    mid = p10 + _lsr16(p00)
    mid2 = p01 + lax.bitwise_and(mid, _i32(0xFFFF))
    return p11 + _lsr16(mid) + _lsr16(mid2)


def _mod100k(x):
    q = lax.shift_right_logical(_mulhi31(x, MG0, MG1), _i32(15))
    return x - q * _i32(100000)


def _hash_idx(lo0):
    a = lax.bitwise_xor(_lsr16(lo0), lo0)
    zero = jnp.zeros_like(lo0)
    hi, lo = _mul_c(zero, a)
    sh, sl = _shr16(hi, lo)
    hi, lo = _mul_c(lax.bitwise_xor(sh, hi), lax.bitwise_xor(sl, lo))
    sh, sl = _shr16(hi, lo)
    hi = lax.bitwise_xor(sh, hi)
    lo = lax.bitwise_xor(sl, lo)
    hi_pos = lax.bitwise_and(hi, _i32(0x7FFFFFFF))
    lo_pos = lax.bitwise_and(lo, _i32(0x7FFFFFFF))
    h0 = lax.bitwise_and(hi_pos, _i32(0xFFFF))
    h1 = _lsr16(hi_pos)
    d = _mod100k(h0 * _i32(2103))
    s = (d * _i32(32) + h1 * _i32(10656) + _mod100k(lo_pos)
         + jnp.where(lo < 0, _i32(83648), _i32(0))
         + jnp.where(hi < 0, _i32(75808), _i32(0)))
    return _mod100k(s)


@functools.partial(
    pl.kernel,
    mesh=plsc.VectorSubcoreMesh(core_axis_name="c", subcore_axis_name="s"),
    out_type=jax.ShapeDtypeStruct((N_TOK, DIM), jnp.float32),
    compiler_params=pltpu.CompilerParams(use_tc_tiling_on_sc=False),
    scratch_types=[
        pltpu.VMEM((TOK_W,), jnp.int32),
        pltpu.VMEM((KH, NBLK, BLK), jnp.int32),
        pltpu.VMEM((2, KH, BLK, DIM), jnp.float32),
        pltpu.VMEM((2, BLK, DIM), jnp.float32),
        pltpu.SemaphoreType.DMA,
        pltpu.SemaphoreType.DMA,
    ],
)
def _sc_embed(t_hbm, w_hbm, out_hbm, t_v, idx_v, gbuf, acc, gsem, osem):
    wid = lax.axis_index("s") * NC + lax.axis_index("c")
    base = wid * TOK_W

    pltpu.sync_copy(t_hbm.at[pl.ds(base, TOK_W)], t_v)

    def hash_body(ci, _):
        tv = t_v[pl.ds(ci * 16, 16)]
        bb = lax.shift_right_logical(ci, _i32(3))
        kk = lax.bitwise_and(ci, _i32(7))
        for r in range(KH):
            idx_v[_i32(r), bb, pl.ds(kk * 16, 16)] = _hash_idx(tv + _i32(r))
        return 0

    lax.fori_loop(_i32(0), _i32(TOK_W // 16), hash_body, 0)

    def gathers(b, slot):
        for r in range(KH):
            pltpu.async_copy(w_hbm.at[idx_v.at[_i32(r), b]],
                             gbuf.at[slot, _i32(r)], gsem)

    def wait_gathers(b, slot):
        for r in range(KH):
            pltpu.make_async_copy(w_hbm.at[idx_v.at[_i32(r), b]],
                                  gbuf.at[slot, _i32(r)], gsem).wait()

    def out_copy(b, slot):
        pltpu.async_copy(
            acc.at[slot], out_hbm.at[pl.ds(base + b * BLK, BLK)], osem)

    def wait_out_copy(b, slot):
        pltpu.make_async_copy(
            acc.at[slot], out_hbm.at[pl.ds(base + b * BLK, BLK)],
            osem).wait()

    gathers(_i32(0), _i32(0))

    def block_body(b, _):
        slot = lax.bitwise_and(b, _i32(1))
        nslot = lax.bitwise_xor(slot, _i32(1))
        wait_gathers(b, slot)

        @pl.when(b + 1 < NBLK)
        def _():
            gathers(b + 1, nslot)

        @pl.when(b >= 2)
        def _():
            wait_out_copy(b - 2, slot)

        def reduce_body(i, _):
            for h in (0, 16):
                s = pl.ds(h, 16)
                v = (gbuf[slot, _i32(0), i, s] + gbuf[slot, _i32(1), i, s]
                     + gbuf[slot, _i32(2), i, s] + gbuf[slot, _i32(3), i, s])
                acc[slot, i, s] = v * 0.25
            return 0

        lax.fori_loop(_i32(0), _i32(BLK), reduce_body, 0)
        out_copy(b, slot)
        return 0

    lax.fori_loop(_i32(0), _i32(NBLK), block_body, 0)
    wait_out_copy(_i32(NBLK - 2), _i32(0))
    wait_out_copy(_i32(NBLK - 1), _i32(1))


@jax.jit
def kernel(t, W):
    t32 = t.reshape(-1).astype(jnp.int32)
    out = _sc_embed(t32, W)
    return out.reshape(t.shape + (DIM,))


# speedup vs baseline: 16.1866x; 1.2056x over previous
import functools

import jax
import jax.numpy as jnp
from jax import lax
from jax.experimental import pallas as pl
from jax.experimental.pallas import tpu as pltpu
from jax.experimental.pallas import tpu_sc as plsc

NUM_ROWS = 100000
DIM = 32
KH = 4
HASH_C = 73244475
C0 = HASH_C & 0xFFFF
C1 = HASH_C >> 16

MAGIC = (1 << 47) // 100000 + 1
MG0 = MAGIC & 0xFFFF
MG1 = MAGIC >> 16

NC = 2
NS = 16
NW = NC * NS

N_TOK = 16384 * 26
TOK_W = N_TOK // NW
BLK = 128
NBLK = TOK_W // BLK


def _i32(x):
    return jnp.int32(x)


def _lsr16(x):
    return lax.shift_right_logical(x, _i32(16))


def _mul_c(hi, lo):
    a0 = lax.bitwise_and(lo, _i32(0xFFFF))
    a1 = _lsr16(lo)
    p00 = a0 * _i32(C0)
    p01 = a0 * _i32(C1)
    p10 = a1 * _i32(C0)
    p11 = a1 * _i32(C1)
    mid = p10 + _lsr16(p00)
    mid2 = p01 + lax.bitwise_and(mid, _i32(0xFFFF))
    lo_out = lax.bitwise_or(lax.shift_left(mid2, _i32(16)),
                            lax.bitwise_and(p00, _i32(0xFFFF)))
    high = p11 + _lsr16(mid) + _lsr16(mid2)
    hi_out = hi * _i32(HASH_C) + high
    return hi_out, lo_out


def _shr16(hi, lo):
    new_lo = lax.bitwise_or(_lsr16(lo), lax.shift_left(hi, _i32(16)))
    return lax.shift_right_arithmetic(hi, _i32(16)), new_lo


def _mulhi31(x, m0, m1):
    a0 = lax.bitwise_and(x, _i32(0xFFFF))
    a1 = _lsr16(x)
    p00 = a0 * _i32(m0)
    p01 = a0 * _i32(m1)
    p10 = a1 * _i32(m0)
    p11 = a1 * _i32(m1)
    mid = p10 + _lsr16(p00)
    mid2 = p01 + lax.bitwise_and(mid, _i32(0xFFFF))
    return p11 + _lsr16(mid) + _lsr16(mid2)


def _mod100k(x):
    q = lax.shift_right_logical(_mulhi31(x, MG0, MG1), _i32(15))
    return x - q * _i32(100000)


def _hash_idx(lo0):
    a = lax.bitwise_xor(_lsr16(lo0), lo0)
    zero = jnp.zeros_like(lo0)
    hi, lo = _mul_c(zero, a)
    sh, sl = _shr16(hi, lo)
    hi, lo = _mul_c(lax.bitwise_xor(sh, hi), lax.bitwise_xor(sl, lo))
    sh, sl = _shr16(hi, lo)
    hi = lax.bitwise_xor(sh, hi)
    lo = lax.bitwise_xor(sl, lo)
    hi_pos = lax.bitwise_and(hi, _i32(0x7FFFFFFF))
    lo_pos = lax.bitwise_and(lo, _i32(0x7FFFFFFF))
    h0 = lax.bitwise_and(hi_pos, _i32(0xFFFF))
    h1 = _lsr16(hi_pos)
    d = _mod100k(h0 * _i32(2103))
    s = (d * _i32(32) + h1 * _i32(10656) + _mod100k(lo_pos)
         + jnp.where(lo < 0, _i32(83648), _i32(0))
         + jnp.where(hi < 0, _i32(75808), _i32(0)))
    return _mod100k(s)


@functools.partial(
    pl.kernel,
    mesh=plsc.VectorSubcoreMesh(core_axis_name="c", subcore_axis_name="s"),
    out_type=jax.ShapeDtypeStruct((N_TOK, DIM), jnp.float32),
    compiler_params=pltpu.CompilerParams(use_tc_tiling_on_sc=False),
    scratch_types=[
        pltpu.VMEM((TOK_W,), jnp.int32),
        pltpu.VMEM((KH, NBLK, BLK), jnp.int32),
        pltpu.VMEM((2, KH, BLK, DIM), jnp.float32),
        pltpu.VMEM((2, BLK, DIM), jnp.float32),
        pltpu.SemaphoreType.DMA,
        pltpu.SemaphoreType.DMA,
    ],
)
def _sc_embed(t_hbm, w_hbm, out_hbm, t_v, idx_v, gbuf, acc, gsem, osem):
    wid = lax.axis_index("s") * NC + lax.axis_index("c")
    base = wid * TOK_W

    pltpu.sync_copy(t_hbm.at[pl.ds(base, TOK_W)], t_v)

    @plsc.parallel_loop(_i32(0), _i32(TOK_W // 16), step=_i32(1), unroll=2)
    def hash_body(ci):
        tv = t_v[pl.ds(ci * 16, 16)]
        bb = lax.shift_right_logical(ci, _i32(3))
        kk = lax.bitwise_and(ci, _i32(7))
        for r in range(KH):
            idx_v[_i32(r), bb, pl.ds(kk * 16, 16)] = _hash_idx(tv + _i32(r))

    def gathers(b, slot):
        for r in range(KH):
            pltpu.async_copy(w_hbm.at[idx_v.at[_i32(r), b]],
                             gbuf.at[slot, _i32(r)], gsem)

    def wait_gathers(b, slot):
        for r in range(KH):
            pltpu.make_async_copy(w_hbm.at[idx_v.at[_i32(r), b]],
                                  gbuf.at[slot, _i32(r)], gsem).wait()

    def out_copy(b, slot):
        pltpu.async_copy(
            acc.at[slot], out_hbm.at[pl.ds(base + b * BLK, BLK)], osem)

    def wait_out_copy(b, slot):
        pltpu.make_async_copy(
            acc.at[slot], out_hbm.at[pl.ds(base + b * BLK, BLK)],
            osem).wait()

    gathers(_i32(0), _i32(0))

    def block_body(b, _):
        slot = lax.bitwise_and(b, _i32(1))
        nslot = lax.bitwise_xor(slot, _i32(1))
        wait_gathers(b, slot)

        @pl.when(b + 1 < NBLK)
        def _():
            gathers(b + 1, nslot)

        @pl.when(b >= 2)
        def _():
            wait_out_copy(b - 2, slot)

        @plsc.parallel_loop(_i32(0), _i32(BLK), step=_i32(1), unroll=8)
        def reduce_body(i):
            for h in (0, 16):
                s = pl.ds(h, 16)
                v = (gbuf[slot, _i32(0), i, s] + gbuf[slot, _i32(1), i, s]
                     + gbuf[slot, _i32(2), i, s] + gbuf[slot, _i32(3), i, s])
                acc[slot, i, s] = v * 0.25

        out_copy(b, slot)
        return 0

    lax.fori_loop(_i32(0), _i32(NBLK), block_body, 0)
    wait_out_copy(_i32(NBLK - 2), _i32(0))
    wait_out_copy(_i32(NBLK - 1), _i32(1))


@jax.jit
def kernel(t, W):
    t32 = t.reshape(-1).astype(jnp.int32)
    out = _sc_embed(t32, W)
    return out.reshape(t.shape + (DIM,))


# speedup vs baseline: 17.5778x; 1.0859x over previous
import functools

import jax
import jax.numpy as jnp
from jax import lax
from jax.experimental import pallas as pl
from jax.experimental.pallas import tpu as pltpu
from jax.experimental.pallas import tpu_sc as plsc

NUM_ROWS = 100000
DIM = 32
KH = 4
HASH_C = 73244475
C0 = HASH_C & 0xFFFF
C1 = HASH_C >> 16

MAGIC = (1 << 47) // 100000 + 1
MG0 = MAGIC & 0xFFFF
MG1 = MAGIC >> 16

NC = 2
NS = 16
NW = NC * NS

N_TOK = 16384 * 26
TOK_W = N_TOK // NW
BLK = 128
NBLK = TOK_W // BLK

HROWS = N_TOK // 128
HBLK = 128
HGRID = HROWS // HBLK

WBLK = 4000
WGRID = NUM_ROWS // WBLK


def _i32(x):
    return jnp.int32(x)


def _lsr16(x):
    return lax.shift_right_logical(x, _i32(16))


def _mul_c(hi, lo):
    a0 = lax.bitwise_and(lo, _i32(0xFFFF))
    a1 = _lsr16(lo)
    p00 = a0 * _i32(C0)
    p01 = a0 * _i32(C1)
    p10 = a1 * _i32(C0)
    p11 = a1 * _i32(C1)
    mid = p10 + _lsr16(p00)
    mid2 = p01 + lax.bitwise_and(mid, _i32(0xFFFF))
    lo_out = lax.bitwise_or(lax.shift_left(mid2, _i32(16)),
                            lax.bitwise_and(p00, _i32(0xFFFF)))
    high = p11 + _lsr16(mid) + _lsr16(mid2)
    hi_out = hi * _i32(HASH_C) + high
    return hi_out, lo_out


def _shr16(hi, lo):
    new_lo = lax.bitwise_or(_lsr16(lo), lax.shift_left(hi, _i32(16)))
    return lax.shift_right_arithmetic(hi, _i32(16)), new_lo


def _mulhi31(x, m0, m1):
    a0 = lax.bitwise_and(x, _i32(0xFFFF))
    a1 = _lsr16(x)
    p00 = a0 * _i32(m0)
    p01 = a0 * _i32(m1)
    p10 = a1 * _i32(m0)
    p11 = a1 * _i32(m1)
    mid = p10 + _lsr16(p00)
    mid2 = p01 + lax.bitwise_and(mid, _i32(0xFFFF))
    return p11 + _lsr16(mid) + _lsr16(mid2)


def _mod100k(x):
    q = lax.shift_right_logical(_mulhi31(x, MG0, MG1), _i32(15))
    return x - q * _i32(100000)


def _hash_idx(lo0):
    a = lax.bitwise_xor(_lsr16(lo0), lo0)
    zero = jnp.zeros_like(lo0)
    hi, lo = _mul_c(zero, a)
    sh, sl = _shr16(hi, lo)
    hi, lo = _mul_c(lax.bitwise_xor(sh, hi), lax.bitwise_xor(sl, lo))
    sh, sl = _shr16(hi, lo)
    hi = lax.bitwise_xor(sh, hi)
    lo = lax.bitwise_xor(sl, lo)
    hi_pos = lax.bitwise_and(hi, _i32(0x7FFFFFFF))
    lo_pos = lax.bitwise_and(lo, _i32(0x7FFFFFFF))
    h0 = lax.bitwise_and(hi_pos, _i32(0xFFFF))
    h1 = _lsr16(hi_pos)
    d = _mod100k(h0 * _i32(2103))
    s = (d * _i32(32) + h1 * _i32(10656) + _mod100k(lo_pos)
         + jnp.where(lo < 0, _i32(83648), _i32(0))
         + jnp.where(hi < 0, _i32(75808), _i32(0)))
    return _mod100k(s)


def _tc_hash_body(t_ref, idx_ref):
    x = t_ref[...]
    for r in range(KH):
        idx_ref[r] = _hash_idx(x + _i32(r))


_tc_hash = pl.pallas_call(
    _tc_hash_body,
    grid=(HGRID,),
    in_specs=[pl.BlockSpec((HBLK, 128), lambda i: (i, _i32(0)))],
    out_specs=pl.BlockSpec((KH, HBLK, 128),
                          lambda i: (_i32(0), i, _i32(0))),
    out_shape=jax.ShapeDtypeStruct((KH, HROWS, 128), jnp.int32),
)


@functools.partial(
    pl.kernel,
    mesh=plsc.VectorSubcoreMesh(core_axis_name="c", subcore_axis_name="s"),
    out_type=jax.ShapeDtypeStruct((N_TOK, DIM), jnp.float32),
    compiler_params=pltpu.CompilerParams(use_tc_tiling_on_sc=False),
    scratch_types=[
        pltpu.VMEM((KH, TOK_W), jnp.int32),
        pltpu.VMEM((2, KH, BLK, DIM), jnp.float32),
        pltpu.VMEM((2, BLK, DIM), jnp.float32),
        pltpu.SemaphoreType.DMA,
        pltpu.SemaphoreType.DMA,
    ],
)
def _sc_embed(idx_hbm, wb_hbm, out_hbm, idx_v, gbuf, acc, gsem, osem):
    wid = lax.axis_index("s") * NC + lax.axis_index("c")
    base = wid * TOK_W

    for r in range(KH):
        pltpu.sync_copy(idx_hbm.at[_i32(r), pl.ds(base, TOK_W)],
                        idx_v.at[_i32(r)])

    def gathers(b, slot):
        for r in range(KH):
            pltpu.async_copy(
                wb_hbm.at[idx_v.at[_i32(r), pl.ds(b * BLK, BLK)]],
                gbuf.at[slot, _i32(r)], gsem)

    def wait_gathers(b, slot):
        for r in range(KH):
            pltpu.make_async_copy(
                wb_hbm.at[idx_v.at[_i32(r), pl.ds(b * BLK, BLK)]],
                gbuf.at[slot, _i32(r)], gsem).wait()

    def out_copy(b, slot):
        pltpu.async_copy(
            acc.at[slot], out_hbm.at[pl.ds(base + b * BLK, BLK)], osem)

    def wait_out_copy(b, slot):
        pltpu.make_async_copy(
            acc.at[slot], out_hbm.at[pl.ds(base + b * BLK, BLK)],
            osem).wait()

    gathers(_i32(0), _i32(0))

    def block_body(b, _):
        slot = lax.bitwise_and(b, _i32(1))
        nslot = lax.bitwise_xor(slot, _i32(1))
        wait_gathers(b, slot)

        @pl.when(b + 1 < NBLK)
        def _():
            gathers(b + 1, nslot)

        @pl.when(b >= 2)
        def _():
            wait_out_copy(b - 2, slot)

        @plsc.parallel_loop(_i32(0), _i32(BLK), step=_i32(1), unroll=8)
        def reduce_body(i):
            for h in (0, 16):
                s = pl.ds(h, 16)
                v = (gbuf[slot, _i32(0), i, s] + gbuf[slot, _i32(1), i, s]
                     + gbuf[slot, _i32(2), i, s] + gbuf[slot, _i32(3), i, s])
                acc[slot, i, s] = v * 0.25

        out_copy(b, slot)
        return 0

    lax.fori_loop(_i32(0), _i32(NBLK), block_body, 0)
    wait_out_copy(_i32(NBLK - 2), _i32(0))
    wait_out_copy(_i32(NBLK - 1), _i32(1))


@jax.jit
def kernel(t, W):
    t32 = t.reshape(-1).astype(jnp.int32).reshape(HROWS, 128)
    idx = _tc_hash(t32)
    out = _sc_embed(idx.reshape(KH, N_TOK), W)
    return out.reshape(t.shape + (DIM,))
